# baseline (speedup 1.0000x reference)
import functools

import jax
import jax.numpy as jnp
from jax import lax
from jax.experimental import pallas as pl
from jax.experimental.pallas import tpu as pltpu
from jax.experimental.pallas import tpu_sc as plsc

_EPS = 1e-5
_TR = 512


def _tables_body(tokt_ref, segt_ref, post_ref, aux_ref, T_ref, stats_ref):
    R = _TR
    tokt = tokt_ref[...]
    segt = segt_ref[...]
    post = post_ref[...]
    num_w = aux_ref[0:1, :]
    num_b = aux_ref[1:2, :]
    gamma = aux_ref[2:3, :]
    beta = aux_ref[3:4, :]
    ridx = jax.lax.broadcasted_iota(jnp.int32, (R, 1), 0)

    def oh(idx, n):
        return (idx == jax.lax.broadcasted_iota(jnp.int32, (R, n), 1)
                ).astype(jnp.float32)

    dot = functools.partial(jnp.dot, preferred_element_type=jnp.float32)

    v = ridx // 84
    s = (ridx // 4) % 21
    p = ridx % 4
    E = dot(oh(v, 8), tokt) + dot(oh(s, 24), segt) + dot(oh(p, 24), post)
    muE = jnp.mean(E, axis=1, keepdims=True)
    varE = jnp.mean((E - muE) ** 2, axis=1, keepdims=True)
    text_rows = (E - muE) * jax.lax.rsqrt(varE + _EPS) * gamma + beta

    j = ridx - 420
    s2 = j // 4
    p2 = jnp.where(j >= 0, j % 4, -1)
    c = num_b + dot(oh(s2, 24), segt) + dot(oh(p2, 24), post)
    muC = jnp.mean(c, axis=1, keepdims=True)
    num_rows = (c - muC) * gamma

    mu_w = jnp.mean(num_w, axis=1, keepdims=True)
    var_w = jnp.mean((num_w - mu_w) ** 2, axis=1, keepdims=True)
    a_row = (num_w - mu_w) * gamma

    isnum_row = jnp.logical_and(ridx >= 420, ridx < 504)
    T = jnp.where(isnum_row, num_rows, text_rows)
    T = jnp.where(ridx == 504, a_row, T)
    T = jnp.where(ridx == 505, beta, T)
    T_ref[...] = T

    ridx2 = jax.lax.broadcasted_iota(jnp.int32, (128, 1), 0)

    def oh2(idx, n):
        return (idx == jax.lax.broadcasted_iota(jnp.int32, (128, n), 1)
                ).astype(jnp.float32)

    s3 = ridx2 // 4
    p3 = ridx2 % 4
    c2 = num_b + dot(oh2(s3, 24), segt) + dot(oh2(p3, 24), post)
    muC2 = jnp.mean(c2, axis=1, keepdims=True)
    var_c = jnp.mean((c2 - muC2) ** 2, axis=1, keepdims=True)
    cov_wc = jnp.mean((num_w - mu_w) * (c2 - muC2), axis=1, keepdims=True)
    z = jnp.zeros((128, 1), jnp.float32)
    stats_ref[...] = jnp.concatenate(
        [var_c, cov_wc, z + var_w, z, z, z, z, z], axis=1)


def _coef_body(idx_ref, isn_ref, vals_ref, stats_ref,
               sA_ref, sB_ref, sC_ref):
    TB = idx_ref.shape[2]
    idx = idx_ref[0]
    isn = (isn_ref[0] != 0).astype(jnp.float32)
    vals = vals_ref[0]
    ohT = (jax.lax.broadcasted_iota(jnp.int32, (128, TB), 0) == idx
           ).astype(jnp.float32)
    g = jax.lax.dot_general(stats_ref[...], ohT, (((0,), (0,)), ((), ())),
                            preferred_element_type=jnp.float32)
    varc = g[0:1, :]
    covwc = g[1:2, :]
    varw = g[2:3, :]
    var = vals * vals * varw + 2.0 * vals * covwc + varc
    rstd = jax.lax.rsqrt(var + _EPS)
    sA_ref[0] = vals * rstd * isn
    sB_ref[0] = isn * rstd + (1.0 - isn)
    sC_ref[0] = isn


def _make_sc_kernel(B, L, D, NC, NS):
    NW = NC * NS
    C = L
    GC = 24
    per_wb = B // NW
    per_w = per_wb * L
    nchunk = per_wb
    NPEEL = nchunk % 3
    mesh = plsc.VectorSubcoreMesh(core_axis_name="c", subcore_axis_name="s")

    @functools.partial(
        pl.kernel,
        out_type=jax.ShapeDtypeStruct((B, L, D), jnp.float32),
        mesh=mesh,
        scratch_types=[
            pltpu.VMEM((nchunk, GC), jnp.int32),
            pltpu.VMEM((per_w + 16,), jnp.float32),
            pltpu.VMEM((per_w + 16,), jnp.float32),
            pltpu.VMEM((per_w + 16,), jnp.float32),
            pltpu.VMEM((2, D), jnp.float32),
            pltpu.VMEM((GC, D), jnp.float32),
            pltpu.VMEM((GC, D), jnp.float32),
            pltpu.VMEM((GC, D), jnp.float32),
            pltpu.VMEM((1, L, D), jnp.float32),
            pltpu.VMEM((1, L, D), jnp.float32),
            pltpu.SemaphoreType.DMA,
            pltpu.SemaphoreType.DMA,
            pltpu.SemaphoreType.DMA,
            pltpu.SemaphoreType.DMA,
            pltpu.SemaphoreType.DMA,
        ],
    )
    def sc_kernel(T_hbm, row_hbm, sA_hbm, sB_hbm, sC_hbm, out_hbm,
                  row_v, sA_v, sB_v, sC_v, ab_v, buf0, buf1, buf2,
                  st0, st1, g0, g1, g2, o0, o1):
        wid = lax.axis_index("s") * NC + lax.axis_index("c")
        bbase = wid * per_wb
        pltpu.sync_copy(row_hbm.at[wid], row_v)
        pltpu.sync_copy(sA_hbm.at[wid], sA_v.at[pl.ds(0, per_w)])
        pltpu.sync_copy(sB_hbm.at[wid], sB_v.at[pl.ds(0, per_w)])
        pltpu.sync_copy(sC_hbm.at[wid], sC_v.at[pl.ds(0, per_w)])
        pltpu.sync_copy(T_hbm.at[pl.ds(504, 2)], ab_v)

        bufs = (buf0, buf1, buf2)
        sts = (st0, st1)
        gsems = (g0, g1, g2)
        osems = (o0, o1)

        def start_gather(r, b):
            pltpu.async_copy(T_hbm.at[row_v.at[r]], bufs[b], gsems[b])

        def wait_gather(r, b):
            pltpu.make_async_copy(T_hbm.at[row_v.at[r]], bufs[b],
                                  gsems[b]).wait()

        def out_slice(r):
            return out_hbm.at[pl.ds(bbase + r, 1)]

        def start_out(r, s):
            pltpu.async_copy(sts[s], out_slice(r), osems[s])

        def wait_out(r, s):
            pltpu.make_async_copy(sts[s], out_slice(r), osems[s]).wait()

        def compute(r, b, s):
            buf = bufs[b]
            st = sts[s]
            coff = r * C
            half = D // 32
            for jh in range(2):
                sls = [pl.ds((jh * half + k) * 16, 16) for k in range(half)]
                aj = [ab_v[0, sl] for sl in sls]
                bj = [ab_v[1, sl] for sl in sls]

                def body(t, _, sls=sls, aj=aj, bj=bj):
                    sa = sA_v[pl.ds(coff + t, 16)][0]
                    sb = sB_v[pl.ds(coff + t, 16)][0]
                    sc = sC_v[pl.ds(coff + t, 16)][0]
                    for k, sl in enumerate(sls):
                        st[0, t, sl] = (sa * aj[k] + sb * buf[t, sl]
                                        + sc * bj[k])
                    return 0

                lax.fori_loop(0, C, body, 0)

        def step(r, b, s):
            wait_gather(r, b)

            @pl.when(r >= 2)
            def _():
                wait_out(r - 2, s)

            compute(r, b, s)
            start_out(r, s)

            @pl.when(r + 2 < nchunk)
            def _():
                start_gather(r + 2, (b + 2) % 3)

        start_gather(0, 0)
        start_gather(1, 1)

        def group(gidx, _):
            for b in range(3):
                r6 = gidx * 6 + b
                step(r6, b, b % 2)
            for b in range(3):
                r6 = gidx * 6 + 3 + b
                step(r6, b, (3 + b) % 2)
            return 0

        lax.fori_loop(0, (nchunk - NPEEL) // 6, group, 0)
        for k in range(NPEEL):
            r = nchunk - NPEEL + k
            step(r, r % 3, r % 2)
        wait_out(nchunk - 2, (nchunk - 2) % 2)
        wait_out(nchunk - 1, (nchunk - 1) % 2)

    return sc_kernel


def kernel(token_ids, is_number, number_vals, segment_ids, pos_ids,
           token_table, num_w, num_b, seg_table, pos_table, gamma, beta):
    B, L = token_ids.shape
    V, D = token_table.shape
    S = seg_table.shape[0]
    N = B * L

    tokt = jnp.pad(token_table, ((0, 8 - V), (0, 0)))
    segt = jnp.pad(seg_table, ((0, -S % 8), (0, 0)))
    post = jnp.pad(pos_table, ((0, -S % 8), (0, 0)))
    aux = jnp.concatenate([
        jnp.stack([num_w, num_b, gamma, beta]),
        jnp.zeros((4, D), jnp.float32)], axis=0)
    full = lambda r, c: pl.BlockSpec((r, c), lambda: (0, 0))
    T, stats = pl.pallas_call(
        _tables_body,
        in_specs=[full(8, D), full(24, D), full(24, D), full(8, D)],
        out_specs=[full(_TR, D), full(128, 8)],
        out_shape=[jax.ShapeDtypeStruct((_TR, D), jnp.float32),
                   jax.ShapeDtypeStruct((128, 8), jnp.float32)],
    )(tokt, segt, post, aux)

    TB = 512
    NB = N // TB
    s4p = segment_ids.astype(jnp.int32) * 4 + pos_ids.astype(jnp.int32)
    idxn = s4p.reshape(NB, 1, TB)
    isn3 = is_number.astype(jnp.int32).reshape(NB, 1, TB)
    vals3 = number_vals.astype(jnp.float32).reshape(NB, 1, TB)
    idx_spec = pl.BlockSpec((1, 1, TB), lambda i: (i, 0, 0))
    row_spec = pl.BlockSpec((1, 1, TB), lambda i: (i, 0, 0))
    coef_shape = jax.ShapeDtypeStruct((NB, 1, TB), jnp.float32)
    sA, sB, sC = pl.pallas_call(
        _coef_body,
        grid=(NB,),
        in_specs=[idx_spec, idx_spec, idx_spec,
                  pl.BlockSpec((128, 8), lambda i: (0, 0))],
        out_specs=[row_spec, row_spec, row_spec],
        out_shape=[coef_shape, coef_shape, coef_shape],
        compiler_params=pltpu.CompilerParams(
            dimension_semantics=("arbitrary",)),
    )(idxn, isn3, vals3, stats)

    NC, NS = 2, 16
    NW = NC * NS
    per_wb = B // NW
    per_w = N // NW
    flat_s4p = s4p.reshape(N)
    row = jnp.where(is_number.reshape(N), 420 + flat_s4p,
                    token_ids.astype(jnp.int32).reshape(N) * 84 + flat_s4p)
    row_arr = jnp.pad(row.reshape(NW, per_wb, L),
                      ((0, 0), (0, 0), (0, 24 - L)))
    sA = sA.reshape(NW, per_w)
    sB = sB.reshape(NW, per_w)
    sC = sC.reshape(NW, per_w)

    sc = _make_sc_kernel(B, L, D, NC, NS)
    return sc(T, row_arr, sA, sB, sC)


# speedup vs baseline: 1.5501x; 1.5501x over previous
import functools

import jax
import jax.numpy as jnp
from jax import lax
from jax.experimental import pallas as pl
from jax.experimental.pallas import tpu as pltpu
from jax.experimental.pallas import tpu_sc as plsc

_EPS = 1e-5
_TR = 512


def _tables_body(tokt_ref, segt_ref, post_ref, aux_ref, T_ref, stats_ref):
    R = _TR
    tokt = tokt_ref[...]
    segt = segt_ref[...]
    post = post_ref[...]
    num_w = aux_ref[0:1, :]
    num_b = aux_ref[1:2, :]
    gamma = aux_ref[2:3, :]
    beta = aux_ref[3:4, :]
    ridx = jax.lax.broadcasted_iota(jnp.int32, (R, 1), 0)

    def oh(idx, n):
        return (idx == jax.lax.broadcasted_iota(jnp.int32, (R, n), 1)
                ).astype(jnp.float32)

    dot = functools.partial(jnp.dot, preferred_element_type=jnp.float32)

    v = ridx // 84
    s = (ridx // 4) % 21
    p = ridx % 4
    E = dot(oh(v, 8), tokt) + dot(oh(s, 24), segt) + dot(oh(p, 24), post)
    muE = jnp.mean(E, axis=1, keepdims=True)
    varE = jnp.mean((E - muE) ** 2, axis=1, keepdims=True)
    text_rows = (E - muE) * jax.lax.rsqrt(varE + _EPS) * gamma + beta

    j = ridx - 420
    s2 = j // 4
    p2 = jnp.where(j >= 0, j % 4, -1)
    c = num_b + dot(oh(s2, 24), segt) + dot(oh(p2, 24), post)
    muC = jnp.mean(c, axis=1, keepdims=True)
    num_rows = (c - muC) * gamma

    mu_w = jnp.mean(num_w, axis=1, keepdims=True)
    var_w = jnp.mean((num_w - mu_w) ** 2, axis=1, keepdims=True)
    a_row = (num_w - mu_w) * gamma

    isnum_row = jnp.logical_and(ridx >= 420, ridx < 504)
    T = jnp.where(isnum_row, num_rows, text_rows)
    T = jnp.where(ridx == 504, a_row, T)
    T = jnp.where(ridx == 505, beta, T)
    T_ref[...] = T

    ridx2 = jax.lax.broadcasted_iota(jnp.int32, (128, 1), 0)

    def oh2(idx, n):
        return (idx == jax.lax.broadcasted_iota(jnp.int32, (128, n), 1)
                ).astype(jnp.float32)

    s3 = ridx2 // 4
    p3 = ridx2 % 4
    c2 = num_b + dot(oh2(s3, 24), segt) + dot(oh2(p3, 24), post)
    muC2 = jnp.mean(c2, axis=1, keepdims=True)
    var_c = jnp.mean((c2 - muC2) ** 2, axis=1, keepdims=True)
    cov_wc = jnp.mean((num_w - mu_w) * (c2 - muC2), axis=1, keepdims=True)
    z = jnp.zeros((128, 1), jnp.float32)
    stats_ref[...] = jnp.concatenate(
        [var_c, cov_wc, z + var_w, z, z, z, z, z], axis=1)


def _coef_body(idx_ref, isn_ref, vals_ref, stats_ref,
               sA_ref, sB_ref, sC_ref):
    TB = idx_ref.shape[2]
    idx = idx_ref[0]
    isn = (isn_ref[0] != 0).astype(jnp.float32)
    vals = vals_ref[0]
    ohT = (jax.lax.broadcasted_iota(jnp.int32, (128, TB), 0) == idx
           ).astype(jnp.float32)
    g = jax.lax.dot_general(stats_ref[...], ohT, (((0,), (0,)), ((), ())),
                            preferred_element_type=jnp.float32)
    varc = g[0:1, :]
    covwc = g[1:2, :]
    varw = g[2:3, :]
    var = vals * vals * varw + 2.0 * vals * covwc + varc
    rstd = jax.lax.rsqrt(var + _EPS)
    sA_ref[0] = vals * rstd * isn
    sB_ref[0] = isn * rstd + (1.0 - isn)
    sC_ref[0] = isn


def _make_sc_kernel(N, D, NC, NS, C):
    NW = NC * NS
    per_w = N // NW
    nchunk = per_w // C
    assert per_w % C == 0 and nchunk % 3 == 0
    mesh = plsc.VectorSubcoreMesh(core_axis_name="c", subcore_axis_name="s")

    @functools.partial(
        pl.kernel,
        out_type=jax.ShapeDtypeStruct((N, D), jnp.float32),
        mesh=mesh,
        scratch_types=[
            pltpu.VMEM((nchunk, C), jnp.int32),
            pltpu.VMEM((per_w + 16,), jnp.float32),
            pltpu.VMEM((per_w + 16,), jnp.float32),
            pltpu.VMEM((per_w + 16,), jnp.float32),
            pltpu.VMEM((2, D), jnp.float32),
            pltpu.VMEM((C, D), jnp.float32),
            pltpu.VMEM((C, D), jnp.float32),
            pltpu.VMEM((C, D), jnp.float32),
            pltpu.SemaphoreType.DMA,
            pltpu.SemaphoreType.DMA,
            pltpu.SemaphoreType.DMA,
            pltpu.SemaphoreType.DMA,
            pltpu.SemaphoreType.DMA,
            pltpu.SemaphoreType.DMA,
        ],
    )
    def sc_kernel(T_hbm, row_hbm, sA_hbm, sB_hbm, sC_hbm, out_hbm,
                  row_v, sA_v, sB_v, sC_v, ab_v, buf0, buf1, buf2,
                  g0, g1, g2, o0, o1, o2):
        wid = lax.axis_index("s") * NC + lax.axis_index("c")
        base = wid * per_w
        pltpu.sync_copy(row_hbm.at[wid], row_v)
        pltpu.sync_copy(sA_hbm.at[wid], sA_v.at[pl.ds(0, per_w)])
        pltpu.sync_copy(sB_hbm.at[wid], sB_v.at[pl.ds(0, per_w)])
        pltpu.sync_copy(sC_hbm.at[wid], sC_v.at[pl.ds(0, per_w)])
        pltpu.sync_copy(T_hbm.at[pl.ds(504, 2)], ab_v)

        bufs = (buf0, buf1, buf2)
        gsems = (g0, g1, g2)
        osems = (o0, o1, o2)

        def start_gather(r, b):
            pltpu.async_copy(T_hbm.at[row_v.at[r]], bufs[b], gsems[b])

        def wait_gather(r, b):
            pltpu.make_async_copy(T_hbm.at[row_v.at[r]], bufs[b],
                                  gsems[b]).wait()

        def out_slice(r):
            return out_hbm.at[pl.ds(base + r * C, C)]

        def start_out(r, b):
            pltpu.async_copy(bufs[b], out_slice(r), osems[b])

        def wait_out(r, b):
            pltpu.make_async_copy(bufs[b], out_slice(r), osems[b]).wait()

        def compute(r, b):
            buf = bufs[b]
            coff = r * C
            half = D // 32
            for jh in range(2):
                sls = [pl.ds((jh * half + k) * 16, 16) for k in range(half)]
                aj = [ab_v[0, sl] for sl in sls]
                bj = [ab_v[1, sl] for sl in sls]

                def body(t, _, sls=sls, aj=aj, bj=bj):
                    sa = sA_v[pl.ds(coff + t, 16)][0]
                    sb = sB_v[pl.ds(coff + t, 16)][0]
                    sc = sC_v[pl.ds(coff + t, 16)][0]
                    for k, sl in enumerate(sls):
                        buf[t, sl] = sa * aj[k] + sb * buf[t, sl] + sc * bj[k]
                    return 0

                lax.fori_loop(0, C, body, 0)

        start_gather(0, 0)
        start_gather(1, 1)

        def group(gidx, _):
            for b in range(3):
                r = gidx * 3 + b
                wait_gather(r, b)
                compute(r, b)
                start_out(r, b)

                @pl.when(r >= 1)
                def _():
                    wait_out(r - 1, (b - 1) % 3)

                @pl.when(r + 2 < nchunk)
                def _():
                    start_gather(r + 2, (b + 2) % 3)
            return 0

        lax.fori_loop(0, nchunk // 3, group, 0)
        wait_out(nchunk - 1, (nchunk - 1) % 3)

    return sc_kernel


def kernel(token_ids, is_number, number_vals, segment_ids, pos_ids,
           token_table, num_w, num_b, seg_table, pos_table, gamma, beta):
    B, L = token_ids.shape
    V, D = token_table.shape
    S = seg_table.shape[0]
    N = B * L

    tokt = jnp.pad(token_table, ((0, 8 - V), (0, 0)))
    segt = jnp.pad(seg_table, ((0, -S % 8), (0, 0)))
    post = jnp.pad(pos_table, ((0, -S % 8), (0, 0)))
    aux = jnp.concatenate([
        jnp.stack([num_w, num_b, gamma, beta]),
        jnp.zeros((4, D), jnp.float32)], axis=0)
    full = lambda r, c: pl.BlockSpec((r, c), lambda: (0, 0))
    T, stats = pl.pallas_call(
        _tables_body,
        in_specs=[full(8, D), full(24, D), full(24, D), full(8, D)],
        out_specs=[full(_TR, D), full(128, 8)],
        out_shape=[jax.ShapeDtypeStruct((_TR, D), jnp.float32),
                   jax.ShapeDtypeStruct((128, 8), jnp.float32)],
    )(tokt, segt, post, aux)

    TB = 512
    NB = N // TB
    s4p = segment_ids.astype(jnp.int32) * 4 + pos_ids.astype(jnp.int32)
    idxn = s4p.reshape(NB, 1, TB)
    isn3 = is_number.astype(jnp.int32).reshape(NB, 1, TB)
    vals3 = number_vals.astype(jnp.float32).reshape(NB, 1, TB)
    idx_spec = pl.BlockSpec((1, 1, TB), lambda i: (i, 0, 0))
    row_spec = pl.BlockSpec((1, 1, TB), lambda i: (i, 0, 0))
    coef_shape = jax.ShapeDtypeStruct((NB, 1, TB), jnp.float32)
    sA, sB, sC = pl.pallas_call(
        _coef_body,
        grid=(NB,),
        in_specs=[idx_spec, idx_spec, idx_spec,
                  pl.BlockSpec((128, 8), lambda i: (0, 0))],
        out_specs=[row_spec, row_spec, row_spec],
        out_shape=[coef_shape, coef_shape, coef_shape],
        compiler_params=pltpu.CompilerParams(
            dimension_semantics=("arbitrary",)),
    )(idxn, isn3, vals3, stats)

    NC, NS, C = 2, 16, 32
    NW = NC * NS
    per_w = N // NW
    nchunk = per_w // C
    flat_s4p = s4p.reshape(N)
    row = jnp.where(is_number.reshape(N), 420 + flat_s4p,
                    token_ids.astype(jnp.int32).reshape(N) * 84 + flat_s4p)
    row_arr = row.reshape(NW, nchunk, C)
    sA = sA.reshape(NW, per_w)
    sB = sB.reshape(NW, per_w)
    sC = sC.reshape(NW, per_w)

    sc = _make_sc_kernel(N, D, NC, NS, C)
    out = sc(T, row_arr, sA, sB, sC)
    return out.reshape(B, L, D)
